# initial kernel scaffold (unmeasured)
import jax
import jax.numpy as jnp
from jax import lax
from jax.experimental import pallas as pl
from jax.experimental.pallas import tpu as pltpu


N_CHUNKS = 8


def kernel(A, B):
    M, Ks = A.shape
    _, N = B.shape
    A = A.astype(jnp.bfloat16)
    B = B.astype(jnp.bfloat16)

    def body(a_ref, b_ref, out_ref, recv_ref, send_sem, recv_sem):
        my_x = lax.axis_index("x")
        my_y = lax.axis_index("y")
        peer = (1 - my_x, my_y)

        barrier = pltpu.get_barrier_semaphore()
        pl.semaphore_signal(
            barrier, inc=1, device_id=peer, device_id_type=pl.DeviceIdType.MESH
        )
        pl.semaphore_wait(barrier, 1)

        cn = N // N_CHUNKS
        for i in range(N_CHUNKS):
            out_ref[:, i * cn:(i + 1) * cn] = jnp.dot(
                a_ref[...],
                b_ref[:, i * cn:(i + 1) * cn],
                preferred_element_type=jnp.float32,
            ).astype(jnp.bfloat16)

        rdma = pltpu.make_async_remote_copy(
            src_ref=out_ref,
            dst_ref=recv_ref,
            send_sem=send_sem,
            recv_sem=recv_sem,
            device_id=peer,
            device_id_type=pl.DeviceIdType.MESH,
        )
        rdma.start()
        rdma.wait()

        out_ref[...] = (
            out_ref[...].astype(jnp.float32) + recv_ref[...].astype(jnp.float32)
        ).astype(jnp.bfloat16)

    return pl.pallas_call(
        body,
        out_shape=jax.ShapeDtypeStruct((M, N), jnp.bfloat16),
        in_specs=[
            pl.BlockSpec(memory_space=pltpu.VMEM),
            pl.BlockSpec(memory_space=pltpu.VMEM),
        ],
        out_specs=pl.BlockSpec(memory_space=pltpu.VMEM),
        scratch_shapes=[
            pltpu.VMEM((M, N), jnp.bfloat16),
            pltpu.SemaphoreType.DMA,
            pltpu.SemaphoreType.DMA,
        ],
        compiler_params=pltpu.CompilerParams(collective_id=0),
    )(A, B)


# baseline (device time: 478528 ns/iter reference)
import jax
import jax.numpy as jnp
from jax import lax
from jax.experimental import pallas as pl
from jax.experimental.pallas import tpu as pltpu


N_CHUNKS = 8


def kernel(A, B):
    M, Ks = A.shape
    _, N = B.shape
    m_half = M // 2
    cn = N // N_CHUNKS

    my_y_out = lax.axis_index("y")
    A_half = lax.dynamic_slice_in_dim(A, my_y_out * m_half, m_half, axis=0)
    A_half = A_half.astype(jnp.bfloat16)
    B = B.astype(jnp.bfloat16)

    def body(
        a_ref,
        b_ref,
        out_ref,
        send_buf,
        recv_buf,
        r_buf,
        sx_send, sx_recv, sy_send, sy_recv, lsem,
    ):
        my_x = lax.axis_index("x")
        my_y = lax.axis_index("y")
        x_peer = (1 - my_x, my_y)
        y_peer = (my_x, 1 - my_y)
        rows_me = my_y * m_half

        barrier = pltpu.get_barrier_semaphore()
        for nbr in (x_peer, y_peer):
            pl.semaphore_signal(
                barrier, inc=1, device_id=nbr, device_id_type=pl.DeviceIdType.MESH
            )
        pl.semaphore_wait(barrier, 2)

        y_rdmas = []
        l_copies = []
        for j in range(N_CHUNKS):
            s = j % 2
            send_buf[s] = jnp.dot(
                a_ref[...],
                b_ref[:, j * cn:(j + 1) * cn],
                preferred_element_type=jnp.float32,
            ).astype(jnp.bfloat16)

            xr = pltpu.make_async_remote_copy(
                src_ref=send_buf.at[s],
                dst_ref=recv_buf.at[s],
                send_sem=sx_send.at[j],
                recv_sem=sx_recv.at[j],
                device_id=x_peer,
                device_id_type=pl.DeviceIdType.MESH,
            )
            xr.start()
            xr.wait()

            r_buf[s] = (
                send_buf[s].astype(jnp.float32) + recv_buf[s].astype(jnp.float32)
            ).astype(jnp.bfloat16)

            lc = pltpu.make_async_copy(
                r_buf.at[s],
                out_ref.at[pl.ds(rows_me, m_half), pl.ds(j * cn, cn)],
                lsem.at[j],
            )
            lc.start()
            l_copies.append(lc)
            yr = pltpu.make_async_remote_copy(
                src_ref=r_buf.at[s],
                dst_ref=out_ref.at[pl.ds(rows_me, m_half), pl.ds(j * cn, cn)],
                send_sem=sy_send.at[j],
                recv_sem=sy_recv.at[j],
                device_id=y_peer,
                device_id_type=pl.DeviceIdType.MESH,
            )
            yr.start()
            y_rdmas.append(yr)
            lc.wait()
            yr.wait_send()

        for yr in y_rdmas:
            yr.wait_recv()

    return pl.pallas_call(
        body,
        out_shape=jax.ShapeDtypeStruct((M, N), jnp.bfloat16),
        in_specs=[
            pl.BlockSpec(memory_space=pltpu.VMEM),
            pl.BlockSpec(memory_space=pltpu.VMEM),
        ],
        out_specs=pl.BlockSpec(memory_space=pl.ANY),
        scratch_shapes=[
            pltpu.VMEM((2, m_half, cn), jnp.bfloat16),
            pltpu.VMEM((2, m_half, cn), jnp.bfloat16),
            pltpu.VMEM((2, m_half, cn), jnp.bfloat16),
            pltpu.SemaphoreType.DMA((N_CHUNKS,)),
            pltpu.SemaphoreType.DMA((N_CHUNKS,)),
            pltpu.SemaphoreType.DMA((N_CHUNKS,)),
            pltpu.SemaphoreType.DMA((N_CHUNKS,)),
            pltpu.SemaphoreType.DMA((N_CHUNKS,)),
        ],
        compiler_params=pltpu.CompilerParams(collective_id=0),
    )(A_half, B)


# device time: 266176 ns/iter; 1.7978x vs baseline; 1.7978x over previous
import jax
import jax.numpy as jnp
from jax import lax
from jax.experimental import pallas as pl
from jax.experimental.pallas import tpu as pltpu


N_CHUNKS = 8
RECV_SLOTS = 4


def kernel(A, B):
    M, Ks = A.shape
    _, N = B.shape
    m_half = M // 2
    cn = N // N_CHUNKS

    my_y_out = lax.axis_index("y")
    A_half = lax.dynamic_slice_in_dim(A, my_y_out * m_half, m_half, axis=0)
    A_half = A_half.astype(jnp.bfloat16)
    B = B.astype(jnp.bfloat16)

    def body(
        a_ref,
        b_ref,
        out_ref,
        send_buf,
        recv_buf,
        r_buf,
        sx_send, sx_recv, sy_send, sy_recv, lsem,
    ):
        my_x = lax.axis_index("x")
        my_y = lax.axis_index("y")
        x_peer = (1 - my_x, my_y)
        y_peer = (my_x, 1 - my_y)
        rows_me = my_y * m_half

        barrier = pltpu.get_barrier_semaphore()
        for nbr in (x_peer, y_peer):
            pl.semaphore_signal(
                barrier, inc=1, device_id=nbr, device_id_type=pl.DeviceIdType.MESH
            )
        pl.semaphore_wait(barrier, 2)

        x_rdmas = []
        y_rdmas = []
        l_copies = []

        def process(i):
            t = i % 2
            if i >= 2:
                y_rdmas[i - 2].wait_send()
                l_copies[i - 2].wait()
            x_rdmas[i].wait_recv()
            r_buf[t] = send_buf[i % 2] + recv_buf[i % RECV_SLOTS]
            lc = pltpu.make_async_copy(
                r_buf.at[t],
                out_ref.at[pl.ds(rows_me, m_half), pl.ds(i * cn, cn)],
                lsem.at[i],
            )
            lc.start()
            l_copies.append(lc)
            yr = pltpu.make_async_remote_copy(
                src_ref=r_buf.at[t],
                dst_ref=out_ref.at[pl.ds(rows_me, m_half), pl.ds(i * cn, cn)],
                send_sem=sy_send.at[i],
                recv_sem=sy_recv.at[i],
                device_id=y_peer,
                device_id_type=pl.DeviceIdType.MESH,
            )
            yr.start()
            y_rdmas.append(yr)

        for j in range(N_CHUNKS):
            s = j % 2
            if j >= 2:
                x_rdmas[j - 2].wait_send()
            send_buf[s] = jnp.dot(
                a_ref[...],
                b_ref[:, j * cn:(j + 1) * cn],
                preferred_element_type=jnp.float32,
            ).astype(jnp.bfloat16)
            xr = pltpu.make_async_remote_copy(
                src_ref=send_buf.at[s],
                dst_ref=recv_buf.at[j % RECV_SLOTS],
                send_sem=sx_send.at[j],
                recv_sem=sx_recv.at[j],
                device_id=x_peer,
                device_id_type=pl.DeviceIdType.MESH,
            )
            xr.start()
            x_rdmas.append(xr)
            if j >= 1:
                process(j - 1)
        process(N_CHUNKS - 1)

        for j in (N_CHUNKS - 2, N_CHUNKS - 1):
            x_rdmas[j].wait_send()
            y_rdmas[j].wait_send()
            l_copies[j].wait()
        for yr in y_rdmas:
            yr.wait_recv()

    return pl.pallas_call(
        body,
        out_shape=jax.ShapeDtypeStruct((M, N), jnp.bfloat16),
        in_specs=[
            pl.BlockSpec(memory_space=pltpu.VMEM),
            pl.BlockSpec(memory_space=pltpu.VMEM),
        ],
        out_specs=pl.BlockSpec(memory_space=pl.ANY),
        scratch_shapes=[
            pltpu.VMEM((2, m_half, cn), jnp.bfloat16),
            pltpu.VMEM((RECV_SLOTS, m_half, cn), jnp.bfloat16),
            pltpu.VMEM((2, m_half, cn), jnp.bfloat16),
            pltpu.SemaphoreType.DMA((N_CHUNKS,)),
            pltpu.SemaphoreType.DMA((N_CHUNKS,)),
            pltpu.SemaphoreType.DMA((N_CHUNKS,)),
            pltpu.SemaphoreType.DMA((N_CHUNKS,)),
            pltpu.SemaphoreType.DMA((N_CHUNKS,)),
        ],
        compiler_params=pltpu.CompilerParams(
            collective_id=0,
            vmem_limit_bytes=40 * 1024 * 1024,
        ),
    )(A_half, B)


# device time: 239180 ns/iter; 2.0007x vs baseline; 1.1129x over previous
import jax
import jax.numpy as jnp
from jax import lax
from jax.experimental import pallas as pl
from jax.experimental.pallas import tpu as pltpu


N_CHUNKS = 16
RECV_SLOTS = 4


def kernel(A, B):
    M, Ks = A.shape
    _, N = B.shape
    m_half = M // 2
    cn = N // N_CHUNKS

    my_y_out = lax.axis_index("y")
    A_half = lax.dynamic_slice_in_dim(A, my_y_out * m_half, m_half, axis=0)
    A_half = A_half.astype(jnp.bfloat16)

    def body(
        a_ref,
        b_ref,
        out_ref,
        bd,
        send_buf,
        recv_buf,
        r_buf,
        bsem, sx_send, sx_recv, sy_send, sy_recv, lsem,
    ):
        my_x = lax.axis_index("x")
        my_y = lax.axis_index("y")
        x_peer = (1 - my_x, my_y)
        y_peer = (my_x, 1 - my_y)
        rows_me = my_y * m_half

        def b_fetch(j):
            c = pltpu.make_async_copy(
                b_ref.at[:, pl.ds(j * cn, cn)], bd.at[j % 2], bsem.at[j]
            )
            c.start()
            return c

        b_dmas = [b_fetch(0)]

        barrier = pltpu.get_barrier_semaphore()
        for nbr in (x_peer, y_peer):
            pl.semaphore_signal(
                barrier, inc=1, device_id=nbr, device_id_type=pl.DeviceIdType.MESH
            )
        pl.semaphore_wait(barrier, 2)

        x_rdmas = []
        y_rdmas = []
        l_copies = []

        def process(i):
            t = i % 2
            if i >= 2:
                y_rdmas[i - 2].wait_send()
                l_copies[i - 2].wait()
            x_rdmas[i].wait_recv()
            r_buf[t] = send_buf[i % 2] + recv_buf[i % RECV_SLOTS]
            lc = pltpu.make_async_copy(
                r_buf.at[t],
                out_ref.at[pl.ds(rows_me, m_half), pl.ds(i * cn, cn)],
                lsem.at[i],
            )
            lc.start()
            l_copies.append(lc)
            yr = pltpu.make_async_remote_copy(
                src_ref=r_buf.at[t],
                dst_ref=out_ref.at[pl.ds(rows_me, m_half), pl.ds(i * cn, cn)],
                send_sem=sy_send.at[i],
                recv_sem=sy_recv.at[i],
                device_id=y_peer,
                device_id_type=pl.DeviceIdType.MESH,
            )
            yr.start()
            y_rdmas.append(yr)

        for j in range(N_CHUNKS):
            s = j % 2
            if j + 1 < N_CHUNKS:
                b_dmas.append(b_fetch(j + 1))
            b_dmas[j].wait()
            if j >= 2:
                x_rdmas[j - 2].wait_send()
            send_buf[s] = jnp.dot(
                a_ref[...],
                bd[s].astype(jnp.bfloat16),
                preferred_element_type=jnp.float32,
            ).astype(jnp.bfloat16)
            xr = pltpu.make_async_remote_copy(
                src_ref=send_buf.at[s],
                dst_ref=recv_buf.at[j % RECV_SLOTS],
                send_sem=sx_send.at[j],
                recv_sem=sx_recv.at[j],
                device_id=x_peer,
                device_id_type=pl.DeviceIdType.MESH,
            )
            xr.start()
            x_rdmas.append(xr)
            if j >= 1:
                process(j - 1)
        process(N_CHUNKS - 1)

        for j in (N_CHUNKS - 2, N_CHUNKS - 1):
            x_rdmas[j].wait_send()
            y_rdmas[j].wait_send()
            l_copies[j].wait()
        for yr in y_rdmas:
            yr.wait_recv()

    return pl.pallas_call(
        body,
        out_shape=jax.ShapeDtypeStruct((M, N), jnp.bfloat16),
        in_specs=[
            pl.BlockSpec(memory_space=pltpu.VMEM),
            pl.BlockSpec(memory_space=pl.ANY),
        ],
        out_specs=pl.BlockSpec(memory_space=pl.ANY),
        scratch_shapes=[
            pltpu.VMEM((2, Ks, cn), jnp.float32),
            pltpu.VMEM((2, m_half, cn), jnp.bfloat16),
            pltpu.VMEM((RECV_SLOTS, m_half, cn), jnp.bfloat16),
            pltpu.VMEM((2, m_half, cn), jnp.bfloat16),
            pltpu.SemaphoreType.DMA((N_CHUNKS,)),
            pltpu.SemaphoreType.DMA((N_CHUNKS,)),
            pltpu.SemaphoreType.DMA((N_CHUNKS,)),
            pltpu.SemaphoreType.DMA((N_CHUNKS,)),
            pltpu.SemaphoreType.DMA((N_CHUNKS,)),
            pltpu.SemaphoreType.DMA((N_CHUNKS,)),
        ],
        compiler_params=pltpu.CompilerParams(
            collective_id=0,
            vmem_limit_bytes=48 * 1024 * 1024,
        ),
    )(A_half, B)


# device time: 228269 ns/iter; 2.0963x vs baseline; 1.0478x over previous
import jax
import jax.numpy as jnp
from jax import lax
from jax.experimental import pallas as pl
from jax.experimental.pallas import tpu as pltpu


CHUNKS = [128] + [256] * 15 + [128]
CN_MAX = max(CHUNKS)
OFFS = [sum(CHUNKS[:i]) for i in range(len(CHUNKS))]
N_CHUNKS = len(CHUNKS)
RECV_SLOTS = 4
K_CHUNKS = 4


def kernel(A, B):
    M, Ks = A.shape
    _, N = B.shape
    m_half = M // 2
    kc = Ks // K_CHUNKS
    assert sum(CHUNKS) == N

    def body(
        a_ref,
        b_ref,
        out_ref,
        a_f32,
        a_bf16,
        bd,
        send_buf,
        recv_buf,
        r_buf,
        asem, bsem, sx_send, sx_recv, sy_send, sy_recv, lsem,
    ):
        my_x = lax.axis_index("x")
        my_y = lax.axis_index("y")
        x_peer = (1 - my_x, my_y)
        y_peer = (my_x, 1 - my_y)
        rows_me = my_y * m_half

        def b_fetch(j):
            c = pltpu.make_async_copy(
                b_ref.at[:, pl.ds(OFFS[j], CHUNKS[j])],
                bd.at[j % 2, :, pl.ds(0, CHUNKS[j])],
                bsem.at[j],
            )
            c.start()
            return c

        b_dmas = [b_fetch(0)]
        a_dmas = []
        for k in range(K_CHUNKS):
            c = pltpu.make_async_copy(
                a_ref.at[pl.ds(rows_me, m_half), pl.ds(k * kc, kc)],
                a_f32.at[:, pl.ds(k * kc, kc)],
                asem.at[k],
            )
            c.start()
            a_dmas.append(c)

        barrier = pltpu.get_barrier_semaphore()
        for nbr in (x_peer, y_peer):
            pl.semaphore_signal(
                barrier, inc=1, device_id=nbr, device_id_type=pl.DeviceIdType.MESH
            )
        pl.semaphore_wait(barrier, 2)

        for k in range(K_CHUNKS):
            a_dmas[k].wait()
            a_bf16[:, k * kc:(k + 1) * kc] = a_f32[:, k * kc:(k + 1) * kc].astype(
                jnp.bfloat16
            )

        x_rdmas = []
        y_rdmas = []
        l_copies = []

        def process(i):
            t = i % 2
            ci = CHUNKS[i]
            if i >= 2:
                y_rdmas[i - 2].wait_send()
                l_copies[i - 2].wait()
            x_rdmas[i].wait_recv()
            r_buf[t, :, pl.ds(0, ci)] = (
                send_buf[t, :, pl.ds(0, ci)]
                + recv_buf[i % RECV_SLOTS, :, pl.ds(0, ci)]
            )
            lc = pltpu.make_async_copy(
                r_buf.at[t, :, pl.ds(0, ci)],
                out_ref.at[pl.ds(rows_me, m_half), pl.ds(OFFS[i], ci)],
                lsem.at[i],
            )
            lc.start()
            l_copies.append(lc)
            yr = pltpu.make_async_remote_copy(
                src_ref=r_buf.at[t, :, pl.ds(0, ci)],
                dst_ref=out_ref.at[pl.ds(rows_me, m_half), pl.ds(OFFS[i], ci)],
                send_sem=sy_send.at[i],
                recv_sem=sy_recv.at[i],
                device_id=y_peer,
                device_id_type=pl.DeviceIdType.MESH,
            )
            yr.start()
            y_rdmas.append(yr)

        for j in range(N_CHUNKS):
            s = j % 2
            cj = CHUNKS[j]
            if j + 1 < N_CHUNKS:
                b_dmas.append(b_fetch(j + 1))
            b_dmas[j].wait()
            if j >= 2:
                x_rdmas[j - 2].wait_send()
            send_buf[s, :, pl.ds(0, cj)] = jnp.dot(
                a_bf16[...],
                bd[s, :, pl.ds(0, cj)].astype(jnp.bfloat16),
                preferred_element_type=jnp.float32,
            ).astype(jnp.bfloat16)
            xr = pltpu.make_async_remote_copy(
                src_ref=send_buf.at[s, :, pl.ds(0, cj)],
                dst_ref=recv_buf.at[j % RECV_SLOTS, :, pl.ds(0, cj)],
                send_sem=sx_send.at[j],
                recv_sem=sx_recv.at[j],
                device_id=x_peer,
                device_id_type=pl.DeviceIdType.MESH,
            )
            xr.start()
            x_rdmas.append(xr)
            if j >= 1:
                process(j - 1)
        process(N_CHUNKS - 1)

        for j in (N_CHUNKS - 2, N_CHUNKS - 1):
            x_rdmas[j].wait_send()
            y_rdmas[j].wait_send()
            l_copies[j].wait()
        for yr in y_rdmas:
            yr.wait_recv()

    return pl.pallas_call(
        body,
        out_shape=jax.ShapeDtypeStruct((M, N), jnp.bfloat16),
        in_specs=[
            pl.BlockSpec(memory_space=pl.ANY),
            pl.BlockSpec(memory_space=pl.ANY),
        ],
        out_specs=pl.BlockSpec(memory_space=pl.ANY),
        scratch_shapes=[
            pltpu.VMEM((m_half, Ks), jnp.float32),
            pltpu.VMEM((m_half, Ks), jnp.bfloat16),
            pltpu.VMEM((2, Ks, CN_MAX), jnp.float32),
            pltpu.VMEM((2, m_half, CN_MAX), jnp.bfloat16),
            pltpu.VMEM((RECV_SLOTS, m_half, CN_MAX), jnp.bfloat16),
            pltpu.VMEM((2, m_half, CN_MAX), jnp.bfloat16),
            pltpu.SemaphoreType.DMA((K_CHUNKS,)),
            pltpu.SemaphoreType.DMA((N_CHUNKS,)),
            pltpu.SemaphoreType.DMA((N_CHUNKS,)),
            pltpu.SemaphoreType.DMA((N_CHUNKS,)),
            pltpu.SemaphoreType.DMA((N_CHUNKS,)),
            pltpu.SemaphoreType.DMA((N_CHUNKS,)),
            pltpu.SemaphoreType.DMA((N_CHUNKS,)),
        ],
        compiler_params=pltpu.CompilerParams(
            collective_id=0,
            vmem_limit_bytes=56 * 1024 * 1024,
        ),
    )(A, B)
